# baseline (speedup 1.0000x reference)
import functools

import jax
import jax.numpy as jnp
from jax import lax
from jax.experimental import pallas as pl
from jax.experimental.pallas import tpu as pltpu
from jax.experimental.pallas import tpu_sc as plsc

VOCAB = 1000000
D = 64
B = 4096
SEQ = 200
NCLS = 20

NC = 2
NS = 16
NW = NC * NS
RPW = B // NW
L = 16
C = 32
NK = 8
SEQP = C * NK
KSH = 3
NB = 8
NSTEP = RPW * NK

_MESH = plsc.VectorSubcoreMesh(
    core_axis_name="c", subcore_axis_name="s", num_cores=NC, num_subcores=NS
)


@functools.partial(
    pl.kernel,
    out_type=jax.ShapeDtypeStruct((B, D), jnp.float32),
    mesh=_MESH,
    scratch_types=[
        pltpu.VMEM((RPW, SEQP), jnp.int32),
        pltpu.VMEM((RPW + L,), jnp.int32),
        pltpu.VMEM((NB, C, D), jnp.float32),
        pltpu.VMEM((RPW, D), jnp.float32),
        pltpu.SemaphoreType.DMA((NB,)),
    ],
    compiler_params=pltpu.CompilerParams(use_tc_tiling_on_sc=False),
)
def _pool(x_hbm, len_hbm, tab_hbm, out_hbm, x_v, len_v, buf_v, reps_v, sems):
    wid = lax.axis_index("s") * NC + lax.axis_index("c")
    base = wid * RPW
    pltpu.sync_copy(x_hbm.at[pl.ds(base, RPW)], x_v)
    pltpu.sync_copy(len_hbm.at[pl.ds(base, RPW)], len_v.at[pl.ds(0, RPW)])

    def _len(i):
        return len_v[pl.ds(i, L)][0]

    zeros = jnp.zeros((L,), jnp.float32)

    def _zero(i, _):
        for d in range(D // L):
            reps_v[i, pl.ds(d * L, L)] = zeros
        return 0

    lax.fori_loop(0, RPW, _zero, 0)

    def _step_info(s):
        i = jnp.minimum(s >> KSH, RPW - 1)
        k = s & (NK - 1)
        valid = jnp.logical_and(s < NSTEP, k * C < _len(i))
        return i, k, valid

    def _copy(i, k, b):
        idx = x_v.at[i, pl.ds(k * C, C)]
        return pltpu.make_async_copy(tab_hbm.at[idx], buf_v.at[b], sems.at[b])

    def _start(s, b):
        i, k, valid = _step_info(s)

        @pl.when(valid)
        def _():
            _copy(i, k, b).start()

    def _wait_accum(s, b):
        i, k, valid = _step_info(s)

        @pl.when(valid)
        def _():
            _copy(i, k, b).wait()
            length = _len(i)
            j0 = k * C
            for d in range(D // L):
                acc = reps_v[i, pl.ds(d * L, L)]
                for j in range(C):
                    m = jnp.where(j0 + j < length, 1.0, 0.0).astype(jnp.float32)
                    acc = acc + buf_v[b, j, pl.ds(d * L, L)] * m
                reps_v[i, pl.ds(d * L, L)] = acc

    for b in range(NB):
        _start(jnp.int32(b), b)

    def _group(g, _):
        s0 = g * NB
        for b in range(NB):
            _wait_accum(s0 + b, b)
            _start(s0 + b + NB, b)
        return 0

    lax.fori_loop(0, NSTEP // NB, _group, 0)

    def _finalize(i, _):
        lenf = jnp.full((L,), _len(i), jnp.float32)
        inv = jnp.full((L,), 1.0, jnp.float32) / lenf
        for d in range(D // L):
            reps_v[i, pl.ds(d * L, L)] = reps_v[i, pl.ds(d * L, L)] * inv
        return 0

    lax.fori_loop(0, RPW, _finalize, 0)
    pltpu.sync_copy(reps_v, out_hbm.at[pl.ds(base, RPW)])


_RPK = 2000


def _repack_body(a_ref, b_ref, out_ref):
    out_ref[:, :D] = a_ref[...]
    out_ref[:, D:] = b_ref[...]


def _repack(table):
    nblk = VOCAB // 2 // _RPK
    return pl.pallas_call(
        _repack_body,
        out_shape=jax.ShapeDtypeStruct((VOCAB // 2, 2 * D), jnp.float32),
        grid=(nblk,),
        in_specs=[
            pl.BlockSpec((_RPK, D), lambda i: (i, 0)),
            pl.BlockSpec((_RPK, D), lambda i, n=nblk: (i + n, 0)),
        ],
        out_specs=pl.BlockSpec((_RPK, 2 * D), lambda i: (i, 0)),
    )(table, table)


def _head_body(reps_ref, w_ref, b_ref, out_ref):
    r = jnp.maximum(reps_ref[...], 0.0)
    out_ref[...] = (
        jnp.dot(
            r,
            w_ref[...],
            preferred_element_type=jnp.float32,
            precision=lax.Precision.HIGHEST,
        )
        + b_ref[...]
    )


def _head(reps, W, b2d):
    return pl.pallas_call(
        _head_body,
        out_shape=jax.ShapeDtypeStruct((B, NCLS), jnp.float32),
    )(reps, W, b2d)


def kernel(x, lengths, table, W, b):
    x = x.astype(jnp.int32)
    lengths = lengths.astype(jnp.int32)
    npad = SEQP - SEQ
    filler = (
        jnp.arange(B, dtype=jnp.int32)[:, None] * npad
        + jnp.arange(npad, dtype=jnp.int32)[None, :]
    ) % VOCAB
    xp = jnp.concatenate([x, filler], axis=1)
    half = VOCAB // 2
    xphys = jnp.where(xp < half, 2 * xp, 2 * (xp - half) + 1)
    tdense = _repack(table).reshape(VOCAB, D)
    reps = _pool(xphys, lengths, tdense)
    return _head(reps, W, b.reshape(1, NCLS))


# speedup vs baseline: 1.1483x; 1.1483x over previous
import functools

import jax
import jax.numpy as jnp
from jax import lax
from jax.experimental import pallas as pl
from jax.experimental.pallas import tpu as pltpu
from jax.experimental.pallas import tpu_sc as plsc

VOCAB = 1000000
D = 64
B = 4096
SEQ = 200
NCLS = 20

NC = 2
NS = 16
NW = NC * NS
RPW = B // NW
L = 16
C = 16
NK = 16
SEQP = C * NK
KSH = 4
NB = 8
NSTEP = RPW * NK

ZROWS = 2000
TROWS = VOCAB // 2 + ZROWS

_MESH = plsc.VectorSubcoreMesh(
    core_axis_name="c", subcore_axis_name="s", num_cores=NC, num_subcores=NS
)


@functools.partial(
    pl.kernel,
    out_type=jax.ShapeDtypeStruct((B, D), jnp.float32),
    mesh=_MESH,
    scratch_types=[
        pltpu.VMEM((RPW, SEQP), jnp.int32),
        pltpu.VMEM((RPW, SEQP), jnp.int32),
        pltpu.VMEM((RPW + L,), jnp.int32),
        pltpu.VMEM((NB, C, 2 * D), jnp.float32),
        pltpu.VMEM((RPW, D), jnp.float32),
        pltpu.SemaphoreType.DMA((NB,)),
    ],
)
def _pool(x_hbm, w_hbm, len_hbm, tab_hbm, out_hbm, x_v, w_v, len_v, buf_v,
          reps_v, sems):
    wid = lax.axis_index("s") * NC + lax.axis_index("c")
    base = wid * RPW
    pltpu.sync_copy(x_hbm.at[pl.ds(base, RPW)], x_v)
    pltpu.sync_copy(w_hbm.at[pl.ds(base, RPW)], w_v)
    pltpu.sync_copy(len_hbm.at[pl.ds(base, RPW)], len_v.at[pl.ds(0, RPW)])

    def _len(i):
        return len_v[pl.ds(i, L)][0]

    zeros = jnp.zeros((L,), jnp.float32)

    def _zero(i, _):
        for d in range(D // L):
            reps_v[i, pl.ds(d * L, L)] = zeros
        return 0

    lax.fori_loop(0, RPW, _zero, 0)

    def _step_info(s):
        i = jnp.minimum(s >> KSH, RPW - 1)
        k = s & (NK - 1)
        valid = jnp.logical_and(s < NSTEP, k * C < _len(i))
        return i, k, valid

    def _copy(i, k, b):
        idx = x_v[i, pl.ds(k * C, C)]
        return pltpu.make_async_copy(tab_hbm.at[idx], buf_v.at[b], sems.at[b])

    def _start(s, b):
        i, k, valid = _step_info(s)

        @pl.when(valid)
        def _():
            _copy(i, k, b).start()

    def _wait_accum(s, b):
        i, k, valid = _step_info(s)

        @pl.when(valid)
        def _():
            _copy(i, k, b).wait()
            wv = w_v[i, pl.ds(k * C, C)]
            for d in range(D // L):
                sl = pl.ds(d * L, L)
                vs = [
                    buf_v[b, j, pl.ds(wv[j] + d * L, L)] for j in range(C)
                ]
                while len(vs) > 1:
                    vs = [vs[p] + vs[p + 1] for p in range(0, len(vs) - 1, 2)] + (
                        [vs[-1]] if len(vs) % 2 else []
                    )
                reps_v[i, sl] = reps_v[i, sl] + vs[0]

    for b in range(NB):
        _start(jnp.int32(b), b)

    def _group(g, _):
        s0 = g * NB
        for b in range(NB):
            _wait_accum(s0 + b, b)
            _start(s0 + b + NB, b)
        return 0

    lax.fori_loop(0, NSTEP // NB, _group, 0)

    def _finalize(i, _):
        lenf = jnp.full((L,), _len(i), jnp.float32)
        inv = jnp.full((L,), 1.0, jnp.float32) / lenf
        for d in range(D // L):
            reps_v[i, pl.ds(d * L, L)] = reps_v[i, pl.ds(d * L, L)] * inv
        return 0

    lax.fori_loop(0, RPW, _finalize, 0)
    pltpu.sync_copy(reps_v, out_hbm.at[pl.ds(base, RPW)])


_RPK = 2000
_RBLK = VOCAB // 2 // _RPK


def _repack_body(a_ref, b_ref, out_ref):
    g = pl.program_id(0)

    @pl.when(g < _RBLK)
    def _():
        out_ref[:, :D] = a_ref[...]
        out_ref[:, D:] = b_ref[...]

    @pl.when(g == _RBLK)
    def _():
        out_ref[...] = jnp.zeros((_RPK, 2 * D), jnp.float32)


def _repack(table):
    return pl.pallas_call(
        _repack_body,
        out_shape=jax.ShapeDtypeStruct((TROWS, 2 * D), jnp.float32),
        grid=(_RBLK + 1,),
        in_specs=[
            pl.BlockSpec((_RPK, D), lambda i: (jnp.minimum(i, _RBLK - 1), 0)),
            pl.BlockSpec(
                (_RPK, D), lambda i: (jnp.minimum(i, _RBLK - 1) + _RBLK, 0)
            ),
        ],
        out_specs=pl.BlockSpec((_RPK, 2 * D), lambda i: (i, 0)),
    )(table, table)


def _head_body(reps_ref, w_ref, b_ref, out_ref):
    r = jnp.maximum(reps_ref[...], 0.0)
    out_ref[...] = (
        jnp.dot(
            r,
            w_ref[...],
            preferred_element_type=jnp.float32,
            precision=lax.Precision.HIGHEST,
        )
        + b_ref[...]
    )


def _head(reps, W, b2d):
    return pl.pallas_call(
        _head_body,
        out_shape=jax.ShapeDtypeStruct((B, NCLS), jnp.float32),
    )(reps, W, b2d)


def kernel(x, lengths, table, W, b):
    x = x.astype(jnp.int32)
    lengths = lengths.astype(jnp.int32)
    half = VOCAB // 2
    npad = SEQP - SEQ
    xp = jnp.concatenate([x, jnp.zeros((B, npad), jnp.int32)], axis=1)
    xrow = jnp.where(xp < half, xp, xp - half)
    woff = jnp.where(xp < half, 0, D).astype(jnp.int32)
    pos = jnp.arange(SEQP, dtype=jnp.int32)[None, :]
    flat = pos + SEQP * jnp.arange(B, dtype=jnp.int32)[:, None]
    zidx = half + flat % ZROWS
    xfinal = jnp.where(pos < lengths[:, None], xrow, zidx)
    tdense = _repack(table)
    reps = _pool(xfinal, woff, lengths, tdense)
    return _head(reps, W, b.reshape(1, NCLS))
